# baseline (speedup 1.0000x reference)
import functools

import jax
import jax.numpy as jnp
from jax import lax
from jax.experimental import pallas as pl
from jax.experimental.pallas import tpu as pltpu
from jax.experimental.pallas import tpu_sc as plsc

N = 10000
E = 160000
H = 512
L = 3

NSC = 2
NSUB = 16
NW = NSC * NSUB

NP = 10240
RPT = NP // NSUB

FC = 128
NCHUNK = H // FC

EPT = E // NSUB
EB = 125
NEB = EPT // EB

ES_PER_W = E // NW
ES_B = 125
ES_NB = ES_PER_W // ES_B
ES_JB = 8
ES_NJ = ES_NB // ES_JB


def _chunked(h):
    return jnp.transpose(h.reshape(h.shape[0], NCHUNK, FC), (1, 0, 2))


def _in_proj_body(x_ref, w_ref, b_ref, mw_ref, mb_ref, o_ref, o4_ref):
    acc = jnp.dot(x_ref[...], w_ref[...], preferred_element_type=jnp.float32)
    h = jnp.maximum(acc + b_ref[...], 0.0)
    o_ref[...] = h
    m = jnp.dot(h, mw_ref[...], preferred_element_type=jnp.float32)
    o4_ref[...] = _chunked(m + mb_ref[...])


def _in_proj(x, w, b, mw, mb):
    d_in = x.shape[1]
    r = 2048
    return pl.pallas_call(
        _in_proj_body,
        grid=(NP // r,),
        in_specs=[
            pl.BlockSpec((r, d_in), lambda i: (i, 0)),
            pl.BlockSpec((d_in, H), lambda i: (0, 0)),
            pl.BlockSpec((1, H), lambda i: (0, 0)),
            pl.BlockSpec((H, H), lambda i: (0, 0)),
            pl.BlockSpec((1, H), lambda i: (0, 0)),
        ],
        out_specs=[
            pl.BlockSpec((r, H), lambda i: (i, 0)),
            pl.BlockSpec((NCHUNK, r, FC), lambda i: (0, i, 0)),
        ],
        out_shape=[
            jax.ShapeDtypeStruct((NP, H), jnp.float32),
            jax.ShapeDtypeStruct((NCHUNK, NP, FC), jnp.float32),
        ],
    )(x, w, b.reshape(1, H), mw, mb.reshape(1, H))


def _gru_gates(agg_ref, h_ref, wih_ref, whh_ref, bih_ref, bhh_ref):
    h = h_ref[...]
    gi = jnp.dot(agg_ref[0], wih_ref[0], preferred_element_type=jnp.float32)
    for c in range(1, NCHUNK):
        gi = gi + jnp.dot(agg_ref[c], wih_ref[c],
                          preferred_element_type=jnp.float32)
    gi = gi + bih_ref[...]
    gh = jnp.dot(h, whh_ref[...], preferred_element_type=jnp.float32)
    gh = gh + bhh_ref[...]
    i_r, i_z, i_n = gi[:, :H], gi[:, H:2 * H], gi[:, 2 * H:]
    h_r, h_z, h_n = gh[:, :H], gh[:, H:2 * H], gh[:, 2 * H:]
    rg = jax.nn.sigmoid(i_r + h_r)
    z = jax.nn.sigmoid(i_z + h_z)
    n = jnp.tanh(i_n + rg * h_n)
    return (1.0 - z) * n + z * h


def _gru_body(agg_ref, h_ref, wih_ref, whh_ref, bih_ref, bhh_ref,
              mw_ref, mb_ref, o_ref, o4_ref):
    hn = _gru_gates(agg_ref, h_ref, wih_ref, whh_ref, bih_ref, bhh_ref)
    o_ref[...] = hn
    m = jnp.dot(hn, mw_ref[...], preferred_element_type=jnp.float32)
    o4_ref[...] = _chunked(m + mb_ref[...])


def _gru_final_body(agg_ref, h_ref, wih_ref, whh_ref, bih_ref, bhh_ref,
                    wab_ref, b8_ref, s_ref):
    hn = _gru_gates(agg_ref, h_ref, wih_ref, whh_ref, bih_ref, bhh_ref)
    s_ref[...] = jnp.dot(hn, wab_ref[...],
                         preferred_element_type=jnp.float32) + b8_ref[...]


def _gru(agg4, h, wih4, whh_t, bih, bhh, mw, mb):
    r = 1024
    return pl.pallas_call(
        _gru_body,
        grid=(NP // r,),
        in_specs=[
            pl.BlockSpec((NCHUNK, r, FC), lambda i: (0, i, 0)),
            pl.BlockSpec((r, H), lambda i: (i, 0)),
            pl.BlockSpec((NCHUNK, FC, 3 * H), lambda i: (0, 0, 0)),
            pl.BlockSpec((H, 3 * H), lambda i: (0, 0)),
            pl.BlockSpec((1, 3 * H), lambda i: (0, 0)),
            pl.BlockSpec((1, 3 * H), lambda i: (0, 0)),
            pl.BlockSpec((H, H), lambda i: (0, 0)),
            pl.BlockSpec((1, H), lambda i: (0, 0)),
        ],
        out_specs=[
            pl.BlockSpec((r, H), lambda i: (i, 0)),
            pl.BlockSpec((NCHUNK, r, FC), lambda i: (0, i, 0)),
        ],
        out_shape=[
            jax.ShapeDtypeStruct((NP, H), jnp.float32),
            jax.ShapeDtypeStruct((NCHUNK, NP, FC), jnp.float32),
        ],
    )(agg4, h, wih4, whh_t, bih.reshape(1, 3 * H), bhh.reshape(1, 3 * H),
      mw, mb.reshape(1, H))


def _gru_final(agg4, h, wih4, whh_t, bih, bhh, wab, b8):
    r = 1024
    return pl.pallas_call(
        _gru_final_body,
        grid=(NP // r,),
        in_specs=[
            pl.BlockSpec((NCHUNK, r, FC), lambda i: (0, i, 0)),
            pl.BlockSpec((r, H), lambda i: (i, 0)),
            pl.BlockSpec((NCHUNK, FC, 3 * H), lambda i: (0, 0, 0)),
            pl.BlockSpec((H, 3 * H), lambda i: (0, 0)),
            pl.BlockSpec((1, 3 * H), lambda i: (0, 0)),
            pl.BlockSpec((1, 3 * H), lambda i: (0, 0)),
            pl.BlockSpec((H, 32), lambda i: (0, 0)),
            pl.BlockSpec((1, 32), lambda i: (0, 0)),
        ],
        out_specs=pl.BlockSpec((r, 32), lambda i: (i, 0)),
        out_shape=jax.ShapeDtypeStruct((NP, 32), jnp.float32),
    )(agg4, h, wih4, whh_t, bih.reshape(1, 3 * H), bhh.reshape(1, 3 * H),
      wab, b8)


_MESH = plsc.VectorSubcoreMesh(core_axis_name="c", subcore_axis_name="s",
                               num_cores=NSC, num_subcores=NSUB)


NBUF = 2
HALF = NEB // 2
NGH = HALF // NBUF


@functools.partial(
    pl.kernel,
    out_type=jax.ShapeDtypeStruct((NCHUNK, NP, FC), jnp.float32),
    mesh=_MESH,
    scratch_types=[
        pltpu.VMEM((HALF, EB), jnp.int32),
        pltpu.VMEM((HALF, EB), jnp.int32),
        pltpu.VMEM((NBUF, EB, FC), jnp.float32),
        pltpu.VMEM_SHARED((NP, FC), jnp.float32),
        pltpu.SemaphoreType.DMA,
        pltpu.SemaphoreType.DMA,
        pltpu.SemaphoreType.DMA,
        pltpu.SemaphoreType.DMA,
    ],
)
def _seg_sum_kernel(m4_hbm, src_hbm, dst_hbm, zeros_hbm, out_hbm,
                    src_v, dst_v, rows_v, acc_sh, g0, g1, s0, s1):
    cid = lax.axis_index("c")
    sid = lax.axis_index("s")
    gsems = [g0, g1]
    ssems = [s0, s1]

    def process_chunk(chunk):
        pltpu.sync_copy(zeros_hbm, acc_sh.at[pl.ds(sid * RPT, RPT)])
        plsc.subcore_barrier()

        def start_gather(b, bi):
            pltpu.async_copy(m4_hbm.at[chunk].at[src_v.at[b]],
                             rows_v.at[bi], gsems[bi])

        def wait_gather(b, bi):
            pltpu.make_async_copy(m4_hbm.at[chunk].at[src_v.at[b]],
                                  rows_v.at[bi], gsems[bi]).wait()

        def start_scatter(b, bi):
            pltpu.async_copy(rows_v.at[bi], acc_sh.at[dst_v.at[b]],
                             ssems[bi], add=True)

        def wait_scatter(b, bi):
            pltpu.make_async_copy(rows_v.at[bi], acc_sh.at[dst_v.at[b]],
                                  ssems[bi]).wait()

        for half in range(2):
            pltpu.sync_copy(src_hbm.at[sid].at[pl.ds(half * HALF, HALF)],
                            src_v)
            pltpu.sync_copy(dst_hbm.at[sid].at[pl.ds(half * HALF, HALF)],
                            dst_v)

            for bi in range(NBUF):
                start_gather(bi, bi)

            def body(j, _):
                for bi in range(NBUF):
                    b = j * NBUF + bi
                    wait_gather(b, bi)
                    start_scatter(b, bi)
                for bi in range(NBUF):
                    b = j * NBUF + bi
                    wait_scatter(b, bi)
                    start_gather(b + NBUF, bi)
                return 0

            lax.fori_loop(0, NGH - 1, body, 0)
            for bi in range(NBUF):
                b = (NGH - 1) * NBUF + bi
                wait_gather(b, bi)
                start_scatter(b, bi)
            for bi in range(NBUF):
                wait_scatter((NGH - 1) * NBUF + bi, bi)
        plsc.subcore_barrier()
        pltpu.sync_copy(acc_sh.at[pl.ds(sid * RPT, RPT)],
                        out_hbm.at[chunk].at[pl.ds(sid * RPT, RPT)])
        plsc.subcore_barrier()

    for cc in range(NCHUNK // NSC):
        for c0 in range(NSC):
            chunk = c0 * (NCHUNK // NSC) + cc

            @pl.when(cid == c0)
            def _():
                process_chunk(chunk)


def _seg_sum(m4, src3, dst3, zeros_rows):
    return _seg_sum_kernel(m4, src3, dst3, zeros_rows)


@functools.partial(
    pl.kernel,
    out_type=[
        jax.ShapeDtypeStruct((E, 16), jnp.float32),
        jax.ShapeDtypeStruct((E, 16), jnp.float32),
    ],
    mesh=_MESH,
    compiler_params=pltpu.CompilerParams(use_tc_tiling_on_sc=False),
    scratch_types=[
        pltpu.VMEM((ES_NB, ES_B), jnp.int32),
        pltpu.VMEM((ES_NB, ES_B), jnp.int32),
        pltpu.VMEM((ES_JB * ES_B, 16), jnp.float32),
        pltpu.VMEM((ES_JB * ES_B, 16), jnp.float32),
        pltpu.SemaphoreType.DMA,
        pltpu.SemaphoreType.DMA,
    ],
)
def _edge_gather_kernel(a_hbm, b_hbm, src_hbm, dst_hbm, ga_hbm, gb_hbm,
                        src_v, dst_v, stga_v, stgb_v, asem, bsem):
    cid = lax.axis_index("c")
    sid = lax.axis_index("s")
    wid = sid * NSC + cid
    pltpu.sync_copy(src_hbm.at[wid], src_v)
    pltpu.sync_copy(dst_hbm.at[wid], dst_v)

    def outer(j, _):
        def inner(i, _):
            b = j * ES_JB + i
            cpa = pltpu.async_copy(a_hbm.at[src_v.at[b]],
                                   stga_v.at[pl.ds(i * ES_B, ES_B)], asem)
            cpb = pltpu.async_copy(b_hbm.at[dst_v.at[b]],
                                   stgb_v.at[pl.ds(i * ES_B, ES_B)], bsem)
            cpa.wait()
            cpb.wait()
            return 0

        lax.fori_loop(0, ES_JB, inner, 0)
        base = wid * ES_PER_W + j * (ES_JB * ES_B)
        pltpu.sync_copy(stga_v, ga_hbm.at[pl.ds(base, ES_JB * ES_B)])
        pltpu.sync_copy(stgb_v, gb_hbm.at[pl.ds(base, ES_JB * ES_B)])
        return 0

    lax.fori_loop(0, ES_NJ, outer, 0)


def _combine_body(a_ref, b_ref, o_ref):
    o_ref[...] = jax.nn.sigmoid(a_ref[...] + b_ref[...])


def _combine(ga, gb):
    r = 8000
    out = pl.pallas_call(
        _combine_body,
        grid=(E // r,),
        in_specs=[
            pl.BlockSpec((r, 16), lambda i: (i, 0)),
            pl.BlockSpec((r, 16), lambda i: (i, 0)),
        ],
        out_specs=pl.BlockSpec((r, 16), lambda i: (i, 0)),
        out_shape=jax.ShapeDtypeStruct((E, 16), jnp.float32),
    )(ga, gb)
    return out[:, 0]


def kernel(features, edge_index, W_in, b_in, msg_W, msg_b, gru_Wih, gru_Whh,
           gru_bih, gru_bhh, W_out, b_out):
    src = edge_index[0]
    dst = edge_index[1]
    src3 = src.reshape(NSUB, NEB, EB)
    dst3 = dst.reshape(NSUB, NEB, EB)
    zeros_rows = jnp.zeros((RPT, FC), jnp.float32)

    src_p = src.reshape(NW, ES_NB, ES_B)
    dst_p = dst.reshape(NW, ES_NB, ES_B)

    wih4 = [jnp.transpose(gru_Wih[l]).reshape(NCHUNK, FC, 3 * H)
            for l in range(L)]
    whh_t = [jnp.transpose(gru_Whh[l]) for l in range(L)]
    wab = jnp.zeros((H, 32), jnp.float32)
    wab = wab.at[:, 0].set(W_out[:H, 0]).at[:, 16].set(W_out[H:, 0])
    b32 = jnp.zeros((1, 32), jnp.float32).at[0, 0].set(b_out[0])

    x_p = jnp.zeros((NP, features.shape[1]), jnp.float32).at[:N].set(features)
    h, m4 = _in_proj(x_p, W_in, b_in, msg_W[0], msg_b[0])
    for l in range(L - 1):
        agg4 = _seg_sum(m4, src3, dst3, zeros_rows)
        h, m4 = _gru(agg4, h, wih4[l], whh_t[l], gru_bih[l], gru_bhh[l],
                     msg_W[l + 1], msg_b[l + 1])
    agg4 = _seg_sum(m4, src3, dst3, zeros_rows)
    s = _gru_final(agg4, h, wih4[L - 1], whh_t[L - 1],
                   gru_bih[L - 1], gru_bhh[L - 1], wab, b32)

    a16 = s[:, :16]
    b16 = s[:, 16:]
    ga, gb = _edge_gather_kernel(a16, b16, src_p, dst_p)
    return _combine(ga, gb)


# speedup vs baseline: 1.0125x; 1.0125x over previous
import functools

import jax
import jax.numpy as jnp
from jax import lax
from jax.experimental import pallas as pl
from jax.experimental.pallas import tpu as pltpu
from jax.experimental.pallas import tpu_sc as plsc

N = 10000
E = 160000
H = 512
L = 3

NSC = 2
NSUB = 16
NW = NSC * NSUB

NP = 10240
RPT = NP // NSUB

FC = 128
NCHUNK = H // FC

EPT = E // NSUB
EB = 125
NEB = EPT // EB

ES_PER_W = E // NW
ES_B = 125
ES_NB = ES_PER_W // ES_B
ES_JB = 8
ES_NJ = ES_NB // ES_JB


def _chunked(h):
    return jnp.transpose(h.reshape(h.shape[0], NCHUNK, FC), (1, 0, 2))


def _in_proj_body(x_ref, w_ref, b_ref, mw_ref, mb_ref, o_ref, o4_ref):
    acc = jnp.dot(x_ref[...], w_ref[...], preferred_element_type=jnp.float32)
    h = jnp.maximum(acc + b_ref[...], 0.0)
    o_ref[...] = h
    m = jnp.dot(h, mw_ref[...], preferred_element_type=jnp.float32)
    o4_ref[...] = _chunked(m + mb_ref[...])


def _in_proj(x, w, b, mw, mb):
    d_in = x.shape[1]
    r = 2048
    return pl.pallas_call(
        _in_proj_body,
        grid=(NP // r,),
        in_specs=[
            pl.BlockSpec((r, d_in), lambda i: (i, 0)),
            pl.BlockSpec((d_in, H), lambda i: (0, 0)),
            pl.BlockSpec((1, H), lambda i: (0, 0)),
            pl.BlockSpec((H, H), lambda i: (0, 0)),
            pl.BlockSpec((1, H), lambda i: (0, 0)),
        ],
        out_specs=[
            pl.BlockSpec((r, H), lambda i: (i, 0)),
            pl.BlockSpec((NCHUNK, r, FC), lambda i: (0, i, 0)),
        ],
        out_shape=[
            jax.ShapeDtypeStruct((NP, H), jnp.float32),
            jax.ShapeDtypeStruct((NCHUNK, NP, FC), jnp.float32),
        ],
    )(x, w, b.reshape(1, H), mw, mb.reshape(1, H))


def _gh_body(h_ref, whh_ref, bhh_ref, o_ref):
    acc = jnp.dot(h_ref[...], whh_ref[...], preferred_element_type=jnp.float32)
    o_ref[...] = acc + bhh_ref[...]


def _gh(h, whh_t, bhh):
    r = 2048
    return pl.pallas_call(
        _gh_body,
        grid=(NP // r,),
        in_specs=[
            pl.BlockSpec((r, H), lambda i: (i, 0)),
            pl.BlockSpec((H, 3 * H), lambda i: (0, 0)),
            pl.BlockSpec((1, 3 * H), lambda i: (0, 0)),
        ],
        out_specs=pl.BlockSpec((r, 3 * H), lambda i: (i, 0)),
        out_shape=jax.ShapeDtypeStruct((NP, 3 * H), jnp.float32),
    )(h, whh_t, bhh.reshape(1, 3 * H))


def _gru_gates(agg_ref, h_ref, wih_ref, gh_ref, bih_ref):
    h = h_ref[...]
    gi = jnp.dot(agg_ref[0], wih_ref[0], preferred_element_type=jnp.float32)
    for c in range(1, NCHUNK):
        gi = gi + jnp.dot(agg_ref[c], wih_ref[c],
                          preferred_element_type=jnp.float32)
    gi = gi + bih_ref[...]
    gh = gh_ref[...]
    i_r, i_z, i_n = gi[:, :H], gi[:, H:2 * H], gi[:, 2 * H:]
    h_r, h_z, h_n = gh[:, :H], gh[:, H:2 * H], gh[:, 2 * H:]
    rg = jax.nn.sigmoid(i_r + h_r)
    z = jax.nn.sigmoid(i_z + h_z)
    n = jnp.tanh(i_n + rg * h_n)
    return (1.0 - z) * n + z * h


def _gru_body(agg_ref, h_ref, wih_ref, gh_ref, bih_ref,
              mw_ref, mb_ref, o_ref, o4_ref):
    hn = _gru_gates(agg_ref, h_ref, wih_ref, gh_ref, bih_ref)
    o_ref[...] = hn
    m = jnp.dot(hn, mw_ref[...], preferred_element_type=jnp.float32)
    o4_ref[...] = _chunked(m + mb_ref[...])


def _gru_final_body(agg_ref, h_ref, wih_ref, gh_ref, bih_ref,
                    wab_ref, b8_ref, s_ref):
    hn = _gru_gates(agg_ref, h_ref, wih_ref, gh_ref, bih_ref)
    s_ref[...] = jnp.dot(hn, wab_ref[...],
                         preferred_element_type=jnp.float32) + b8_ref[...]


def _gru(agg4, h, wih4, gh, bih, mw, mb):
    r = 1024
    return pl.pallas_call(
        _gru_body,
        grid=(NP // r,),
        in_specs=[
            pl.BlockSpec((NCHUNK, r, FC), lambda i: (0, i, 0)),
            pl.BlockSpec((r, H), lambda i: (i, 0)),
            pl.BlockSpec((NCHUNK, FC, 3 * H), lambda i: (0, 0, 0)),
            pl.BlockSpec((r, 3 * H), lambda i: (i, 0)),
            pl.BlockSpec((1, 3 * H), lambda i: (0, 0)),
            pl.BlockSpec((H, H), lambda i: (0, 0)),
            pl.BlockSpec((1, H), lambda i: (0, 0)),
        ],
        out_specs=[
            pl.BlockSpec((r, H), lambda i: (i, 0)),
            pl.BlockSpec((NCHUNK, r, FC), lambda i: (0, i, 0)),
        ],
        out_shape=[
            jax.ShapeDtypeStruct((NP, H), jnp.float32),
            jax.ShapeDtypeStruct((NCHUNK, NP, FC), jnp.float32),
        ],
    )(agg4, h, wih4, gh, bih.reshape(1, 3 * H), mw, mb.reshape(1, H))


def _gru_final(agg4, h, wih4, gh, bih, wab, b8):
    r = 1024
    return pl.pallas_call(
        _gru_final_body,
        grid=(NP // r,),
        in_specs=[
            pl.BlockSpec((NCHUNK, r, FC), lambda i: (0, i, 0)),
            pl.BlockSpec((r, H), lambda i: (i, 0)),
            pl.BlockSpec((NCHUNK, FC, 3 * H), lambda i: (0, 0, 0)),
            pl.BlockSpec((r, 3 * H), lambda i: (i, 0)),
            pl.BlockSpec((1, 3 * H), lambda i: (0, 0)),
            pl.BlockSpec((H, 32), lambda i: (0, 0)),
            pl.BlockSpec((1, 32), lambda i: (0, 0)),
        ],
        out_specs=pl.BlockSpec((r, 32), lambda i: (i, 0)),
        out_shape=jax.ShapeDtypeStruct((NP, 32), jnp.float32),
    )(agg4, h, wih4, gh, bih.reshape(1, 3 * H), wab, b8)


_MESH = plsc.VectorSubcoreMesh(core_axis_name="c", subcore_axis_name="s",
                               num_cores=NSC, num_subcores=NSUB)


NBUF = 2
HALF = NEB // 2
NGH = HALF // NBUF


@functools.partial(
    pl.kernel,
    out_type=jax.ShapeDtypeStruct((NCHUNK, NP, FC), jnp.float32),
    mesh=_MESH,
    scratch_types=[
        pltpu.VMEM((HALF, EB), jnp.int32),
        pltpu.VMEM((HALF, EB), jnp.int32),
        pltpu.VMEM((NBUF, EB, FC), jnp.float32),
        pltpu.VMEM_SHARED((NP, FC), jnp.float32),
        pltpu.SemaphoreType.DMA,
        pltpu.SemaphoreType.DMA,
        pltpu.SemaphoreType.DMA,
        pltpu.SemaphoreType.DMA,
    ],
)
def _seg_sum_kernel(m4_hbm, src_hbm, dst_hbm, zeros_hbm, out_hbm,
                    src_v, dst_v, rows_v, acc_sh, g0, g1, s0, s1):
    cid = lax.axis_index("c")
    sid = lax.axis_index("s")
    gsems = [g0, g1]
    ssems = [s0, s1]

    def process_chunk(chunk):
        pltpu.sync_copy(zeros_hbm, acc_sh.at[pl.ds(sid * RPT, RPT)])
        plsc.subcore_barrier()

        def start_gather(b, bi):
            pltpu.async_copy(m4_hbm.at[chunk].at[src_v.at[b]],
                             rows_v.at[bi], gsems[bi])

        def wait_gather(b, bi):
            pltpu.make_async_copy(m4_hbm.at[chunk].at[src_v.at[b]],
                                  rows_v.at[bi], gsems[bi]).wait()

        def start_scatter(b, bi):
            pltpu.async_copy(rows_v.at[bi], acc_sh.at[dst_v.at[b]],
                             ssems[bi], add=True)

        def wait_scatter(b, bi):
            pltpu.make_async_copy(rows_v.at[bi], acc_sh.at[dst_v.at[b]],
                                  ssems[bi]).wait()

        for half in range(2):
            pltpu.sync_copy(src_hbm.at[sid].at[pl.ds(half * HALF, HALF)],
                            src_v)
            pltpu.sync_copy(dst_hbm.at[sid].at[pl.ds(half * HALF, HALF)],
                            dst_v)

            for bi in range(NBUF):
                start_gather(bi, bi)

            def body(j, _):
                for bi in range(NBUF):
                    b = j * NBUF + bi
                    wait_gather(b, bi)
                    start_scatter(b, bi)
                for bi in range(NBUF):
                    b = j * NBUF + bi
                    wait_scatter(b, bi)
                    start_gather(b + NBUF, bi)
                return 0

            lax.fori_loop(0, NGH - 1, body, 0)
            for bi in range(NBUF):
                b = (NGH - 1) * NBUF + bi
                wait_gather(b, bi)
                start_scatter(b, bi)
            for bi in range(NBUF):
                wait_scatter((NGH - 1) * NBUF + bi, bi)
        plsc.subcore_barrier()
        pltpu.sync_copy(acc_sh.at[pl.ds(sid * RPT, RPT)],
                        out_hbm.at[chunk].at[pl.ds(sid * RPT, RPT)])
        plsc.subcore_barrier()

    for cc in range(NCHUNK // NSC):
        for c0 in range(NSC):
            chunk = c0 * (NCHUNK // NSC) + cc

            @pl.when(cid == c0)
            def _():
                process_chunk(chunk)


def _seg_sum(m4, src3, dst3, zeros_rows):
    return _seg_sum_kernel(m4, src3, dst3, zeros_rows)


@functools.partial(
    pl.kernel,
    out_type=[
        jax.ShapeDtypeStruct((E, 16), jnp.float32),
        jax.ShapeDtypeStruct((E, 16), jnp.float32),
    ],
    mesh=_MESH,
    compiler_params=pltpu.CompilerParams(use_tc_tiling_on_sc=False),
    scratch_types=[
        pltpu.VMEM((ES_NB, ES_B), jnp.int32),
        pltpu.VMEM((ES_NB, ES_B), jnp.int32),
        pltpu.VMEM((ES_JB * ES_B, 16), jnp.float32),
        pltpu.VMEM((ES_JB * ES_B, 16), jnp.float32),
        pltpu.SemaphoreType.DMA,
        pltpu.SemaphoreType.DMA,
    ],
)
def _edge_gather_kernel(a_hbm, b_hbm, src_hbm, dst_hbm, ga_hbm, gb_hbm,
                        src_v, dst_v, stga_v, stgb_v, asem, bsem):
    cid = lax.axis_index("c")
    sid = lax.axis_index("s")
    wid = sid * NSC + cid
    pltpu.sync_copy(src_hbm.at[wid], src_v)
    pltpu.sync_copy(dst_hbm.at[wid], dst_v)

    def outer(j, _):
        def inner(i, _):
            b = j * ES_JB + i
            cpa = pltpu.async_copy(a_hbm.at[src_v.at[b]],
                                   stga_v.at[pl.ds(i * ES_B, ES_B)], asem)
            cpb = pltpu.async_copy(b_hbm.at[dst_v.at[b]],
                                   stgb_v.at[pl.ds(i * ES_B, ES_B)], bsem)
            cpa.wait()
            cpb.wait()
            return 0

        lax.fori_loop(0, ES_JB, inner, 0)
        base = wid * ES_PER_W + j * (ES_JB * ES_B)
        pltpu.sync_copy(stga_v, ga_hbm.at[pl.ds(base, ES_JB * ES_B)])
        pltpu.sync_copy(stgb_v, gb_hbm.at[pl.ds(base, ES_JB * ES_B)])
        return 0

    lax.fori_loop(0, ES_NJ, outer, 0)


def _combine_body(a_ref, b_ref, o_ref):
    o_ref[...] = jax.nn.sigmoid(a_ref[...] + b_ref[...])


def _combine(ga, gb):
    r = 8000
    out = pl.pallas_call(
        _combine_body,
        grid=(E // r,),
        in_specs=[
            pl.BlockSpec((r, 16), lambda i: (i, 0)),
            pl.BlockSpec((r, 16), lambda i: (i, 0)),
        ],
        out_specs=pl.BlockSpec((r, 16), lambda i: (i, 0)),
        out_shape=jax.ShapeDtypeStruct((E, 16), jnp.float32),
    )(ga, gb)
    return out[:, 0]


def kernel(features, edge_index, W_in, b_in, msg_W, msg_b, gru_Wih, gru_Whh,
           gru_bih, gru_bhh, W_out, b_out):
    src = edge_index[0]
    dst = edge_index[1]
    src3 = src.reshape(NSUB, NEB, EB)
    dst3 = dst.reshape(NSUB, NEB, EB)
    zeros_rows = jnp.zeros((RPT, FC), jnp.float32)

    src_p = src.reshape(NW, ES_NB, ES_B)
    dst_p = dst.reshape(NW, ES_NB, ES_B)

    wih4 = [jnp.transpose(gru_Wih[l]).reshape(NCHUNK, FC, 3 * H)
            for l in range(L)]
    whh_t = [jnp.transpose(gru_Whh[l]) for l in range(L)]
    wab = jnp.zeros((H, 32), jnp.float32)
    wab = wab.at[:, 0].set(W_out[:H, 0]).at[:, 16].set(W_out[H:, 0])
    b32 = jnp.zeros((1, 32), jnp.float32).at[0, 0].set(b_out[0])

    x_p = jnp.zeros((NP, features.shape[1]), jnp.float32).at[:N].set(features)
    h, m4 = _in_proj(x_p, W_in, b_in, msg_W[0], msg_b[0])
    for l in range(L - 1):
        agg4 = _seg_sum(m4, src3, dst3, zeros_rows)
        gh = _gh(h, whh_t[l], gru_bhh[l])
        h, m4 = _gru(agg4, h, wih4[l], gh, gru_bih[l],
                     msg_W[l + 1], msg_b[l + 1])
    agg4 = _seg_sum(m4, src3, dst3, zeros_rows)
    gh = _gh(h, whh_t[L - 1], gru_bhh[L - 1])
    s = _gru_final(agg4, h, wih4[L - 1], gh, gru_bih[L - 1], wab, b32)

    a16 = s[:, :16]
    b16 = s[:, 16:]
    ga, gb = _edge_gather_kernel(a16, b16, src_p, dst_p)
    return _combine(ga, gb)


# speedup vs baseline: 1.1656x; 1.1512x over previous
import functools

import jax
import jax.numpy as jnp
from jax import lax
from jax.experimental import pallas as pl
from jax.experimental.pallas import tpu as pltpu
from jax.experimental.pallas import tpu_sc as plsc

N = 10000
E = 160000
H = 512
L = 3

NSC = 2
NSUB = 16
NW = NSC * NSUB

NP = 10240
RPT = NP // NSUB

FC = 128
NCHUNK = H // FC

EPT = E // NSUB
EB = 125
NEB = EPT // EB

ES_PER_W = E // NW
ES_B = 125
ES_NB = ES_PER_W // ES_B
ES_JB = 8
ES_NJ = ES_NB // ES_JB


def _chunked(h):
    return jnp.transpose(h.reshape(h.shape[0], NCHUNK, FC), (1, 0, 2))


def _in_proj_body(x_ref, w_ref, b_ref, mw_ref, mb_ref, o_ref, o4_ref):
    acc = jnp.dot(x_ref[...], w_ref[...], preferred_element_type=jnp.float32)
    h = jnp.maximum(acc + b_ref[...], 0.0)
    o_ref[...] = h
    m = jnp.dot(h, mw_ref[...], preferred_element_type=jnp.float32)
    o4_ref[...] = _chunked(m + mb_ref[...])


def _in_proj(x, w, b, mw, mb):
    d_in = x.shape[1]
    r = 2048
    return pl.pallas_call(
        _in_proj_body,
        grid=(NP // r,),
        in_specs=[
            pl.BlockSpec((r, d_in), lambda i: (i, 0)),
            pl.BlockSpec((d_in, H), lambda i: (0, 0)),
            pl.BlockSpec((1, H), lambda i: (0, 0)),
            pl.BlockSpec((H, H), lambda i: (0, 0)),
            pl.BlockSpec((1, H), lambda i: (0, 0)),
        ],
        out_specs=[
            pl.BlockSpec((r, H), lambda i: (i, 0)),
            pl.BlockSpec((NCHUNK, r, FC), lambda i: (0, i, 0)),
        ],
        out_shape=[
            jax.ShapeDtypeStruct((NP, H), jnp.float32),
            jax.ShapeDtypeStruct((NCHUNK, NP, FC), jnp.float32),
        ],
    )(x, w, b.reshape(1, H), mw, mb.reshape(1, H))


def _gh_body(h_ref, whh_ref, bhh_ref, o_ref):
    acc = jnp.dot(h_ref[...], whh_ref[...], preferred_element_type=jnp.float32)
    o_ref[...] = acc + bhh_ref[...]


def _gh(h, whh_t, bhh):
    r = 2048
    return pl.pallas_call(
        _gh_body,
        grid=(NP // r,),
        in_specs=[
            pl.BlockSpec((r, H), lambda i: (i, 0)),
            pl.BlockSpec((H, 3 * H), lambda i: (0, 0)),
            pl.BlockSpec((1, 3 * H), lambda i: (0, 0)),
        ],
        out_specs=pl.BlockSpec((r, 3 * H), lambda i: (i, 0)),
        out_shape=jax.ShapeDtypeStruct((NP, 3 * H), jnp.float32),
    )(h, whh_t, bhh.reshape(1, 3 * H))


def _gru_gates(agg_ref, h_ref, wih_ref, gh_ref, bih_ref):
    h = h_ref[...]
    gi = jnp.dot(agg_ref[0], wih_ref[0], preferred_element_type=jnp.float32)
    for c in range(1, NCHUNK):
        gi = gi + jnp.dot(agg_ref[c], wih_ref[c],
                          preferred_element_type=jnp.float32)
    gi = gi + bih_ref[...]
    gh = gh_ref[...]
    i_r, i_z, i_n = gi[:, :H], gi[:, H:2 * H], gi[:, 2 * H:]
    h_r, h_z, h_n = gh[:, :H], gh[:, H:2 * H], gh[:, 2 * H:]
    rg = jax.nn.sigmoid(i_r + h_r)
    z = jax.nn.sigmoid(i_z + h_z)
    n = jnp.tanh(i_n + rg * h_n)
    return (1.0 - z) * n + z * h


def _gru_body(agg_ref, h_ref, wih_ref, gh_ref, bih_ref,
              mw_ref, mb_ref, o_ref, o4_ref):
    hn = _gru_gates(agg_ref, h_ref, wih_ref, gh_ref, bih_ref)
    o_ref[...] = hn
    m = jnp.dot(hn, mw_ref[...], preferred_element_type=jnp.float32)
    o4_ref[...] = _chunked(m + mb_ref[...])


def _gru_final_body(agg_ref, h_ref, wih_ref, gh_ref, bih_ref,
                    wab_ref, b8_ref, s_ref):
    hn = _gru_gates(agg_ref, h_ref, wih_ref, gh_ref, bih_ref)
    s_ref[...] = jnp.dot(hn, wab_ref[...],
                         preferred_element_type=jnp.float32) + b8_ref[...]


def _gru(agg4, h, wih4, gh, bih, mw, mb):
    r = 1024
    return pl.pallas_call(
        _gru_body,
        grid=(NP // r,),
        in_specs=[
            pl.BlockSpec((NCHUNK, r, FC), lambda i: (0, i, 0)),
            pl.BlockSpec((r, H), lambda i: (i, 0)),
            pl.BlockSpec((NCHUNK, FC, 3 * H), lambda i: (0, 0, 0)),
            pl.BlockSpec((r, 3 * H), lambda i: (i, 0)),
            pl.BlockSpec((1, 3 * H), lambda i: (0, 0)),
            pl.BlockSpec((H, H), lambda i: (0, 0)),
            pl.BlockSpec((1, H), lambda i: (0, 0)),
        ],
        out_specs=[
            pl.BlockSpec((r, H), lambda i: (i, 0)),
            pl.BlockSpec((NCHUNK, r, FC), lambda i: (0, i, 0)),
        ],
        out_shape=[
            jax.ShapeDtypeStruct((NP, H), jnp.float32),
            jax.ShapeDtypeStruct((NCHUNK, NP, FC), jnp.float32),
        ],
    )(agg4, h, wih4, gh, bih.reshape(1, 3 * H), mw, mb.reshape(1, H))


def _gru_final(agg4, h, wih4, gh, bih, wab, b8):
    r = 1024
    return pl.pallas_call(
        _gru_final_body,
        grid=(NP // r,),
        in_specs=[
            pl.BlockSpec((NCHUNK, r, FC), lambda i: (0, i, 0)),
            pl.BlockSpec((r, H), lambda i: (i, 0)),
            pl.BlockSpec((NCHUNK, FC, 3 * H), lambda i: (0, 0, 0)),
            pl.BlockSpec((r, 3 * H), lambda i: (i, 0)),
            pl.BlockSpec((1, 3 * H), lambda i: (0, 0)),
            pl.BlockSpec((H, 32), lambda i: (0, 0)),
            pl.BlockSpec((1, 32), lambda i: (0, 0)),
        ],
        out_specs=pl.BlockSpec((r, 32), lambda i: (i, 0)),
        out_shape=jax.ShapeDtypeStruct((NP, 32), jnp.float32),
    )(agg4, h, wih4, gh, bih.reshape(1, 3 * H), wab, b8)


_MESH = plsc.VectorSubcoreMesh(core_axis_name="c", subcore_axis_name="s",
                               num_cores=NSC, num_subcores=NSUB)


NBUF = 2
HALF = NEB // 2
NGH = HALF // NBUF


@functools.partial(
    pl.kernel,
    out_type=jax.ShapeDtypeStruct((NCHUNK, NP, FC), jnp.float32),
    mesh=_MESH,
    scratch_types=[
        pltpu.VMEM((HALF, EB), jnp.int32),
        pltpu.VMEM((HALF, EB), jnp.int32),
        pltpu.VMEM((NBUF, EB, FC), jnp.float32),
        pltpu.VMEM_SHARED((NP, FC), jnp.float32),
        pltpu.SemaphoreType.DMA,
        pltpu.SemaphoreType.DMA,
        pltpu.SemaphoreType.DMA,
        pltpu.SemaphoreType.DMA,
    ],
)
def _seg_sum_kernel(m4_hbm, src_hbm, dst_hbm, zeros_hbm, out_hbm,
                    src_v, dst_v, rows_v, acc_sh, g0, g1, s0, s1):
    cid = lax.axis_index("c")
    sid = lax.axis_index("s")
    gsems = [g0, g1]
    ssems = [s0, s1]

    def process_chunk(chunk):
        pltpu.sync_copy(zeros_hbm, acc_sh.at[pl.ds(sid * RPT, RPT)])
        plsc.subcore_barrier()

        def start_gather(b, bi):
            pltpu.async_copy(m4_hbm.at[chunk].at[src_v.at[b]],
                             rows_v.at[bi], gsems[bi])

        def wait_gather(b, bi):
            pltpu.make_async_copy(m4_hbm.at[chunk].at[src_v.at[b]],
                                  rows_v.at[bi], gsems[bi]).wait()

        def start_scatter(b, bi):
            pltpu.async_copy(rows_v.at[bi], acc_sh.at[dst_v.at[b]],
                             ssems[bi], add=True)

        def wait_scatter(b, bi):
            pltpu.make_async_copy(rows_v.at[bi], acc_sh.at[dst_v.at[b]],
                                  ssems[bi]).wait()

        for half in range(2):
            pltpu.sync_copy(src_hbm.at[sid].at[pl.ds(half * HALF, HALF)],
                            src_v)
            pltpu.sync_copy(dst_hbm.at[sid].at[pl.ds(half * HALF, HALF)],
                            dst_v)

            for bi in range(NBUF):
                start_gather(bi, bi)

            def body(j, _):
                for bi in range(NBUF):
                    b = j * NBUF + bi
                    wait_gather(b, bi)
                    start_scatter(b, bi)
                    wait_scatter(b, bi)
                    start_gather(b + NBUF, bi)
                return 0

            lax.fori_loop(0, NGH - 1, body, 0)
            for bi in range(NBUF):
                b = (NGH - 1) * NBUF + bi
                wait_gather(b, bi)
                start_scatter(b, bi)
            for bi in range(NBUF):
                wait_scatter((NGH - 1) * NBUF + bi, bi)
        plsc.subcore_barrier()
        pltpu.sync_copy(acc_sh.at[pl.ds(sid * RPT, RPT)],
                        out_hbm.at[chunk].at[pl.ds(sid * RPT, RPT)])
        plsc.subcore_barrier()

    for cc in range(NCHUNK // NSC):
        for c0 in range(NSC):
            chunk = c0 * (NCHUNK // NSC) + cc

            @pl.when(cid == c0)
            def _():
                process_chunk(chunk)


def _seg_sum(m4, src3, dst3, zeros_rows):
    return _seg_sum_kernel(m4, src3, dst3, zeros_rows)


@functools.partial(
    pl.kernel,
    out_type=[
        jax.ShapeDtypeStruct((E, 16), jnp.float32),
        jax.ShapeDtypeStruct((E, 16), jnp.float32),
    ],
    mesh=_MESH,
    compiler_params=pltpu.CompilerParams(use_tc_tiling_on_sc=False),
    scratch_types=[
        pltpu.VMEM((ES_NB, ES_B), jnp.int32),
        pltpu.VMEM((ES_NB, ES_B), jnp.int32),
        pltpu.VMEM((ES_JB * ES_B, 16), jnp.float32),
        pltpu.VMEM((ES_JB * ES_B, 16), jnp.float32),
        pltpu.SemaphoreType.DMA,
        pltpu.SemaphoreType.DMA,
    ],
)
def _edge_gather_kernel(a_hbm, b_hbm, src_hbm, dst_hbm, ga_hbm, gb_hbm,
                        src_v, dst_v, stga_v, stgb_v, asem, bsem):
    cid = lax.axis_index("c")
    sid = lax.axis_index("s")
    wid = sid * NSC + cid
    pltpu.sync_copy(src_hbm.at[wid], src_v)
    pltpu.sync_copy(dst_hbm.at[wid], dst_v)

    def outer(j, _):
        def inner(i, _):
            b = j * ES_JB + i
            cpa = pltpu.async_copy(a_hbm.at[src_v.at[b]],
                                   stga_v.at[pl.ds(i * ES_B, ES_B)], asem)
            cpb = pltpu.async_copy(b_hbm.at[dst_v.at[b]],
                                   stgb_v.at[pl.ds(i * ES_B, ES_B)], bsem)
            cpa.wait()
            cpb.wait()
            return 0

        lax.fori_loop(0, ES_JB, inner, 0)
        base = wid * ES_PER_W + j * (ES_JB * ES_B)
        pltpu.sync_copy(stga_v, ga_hbm.at[pl.ds(base, ES_JB * ES_B)])
        pltpu.sync_copy(stgb_v, gb_hbm.at[pl.ds(base, ES_JB * ES_B)])
        return 0

    lax.fori_loop(0, ES_NJ, outer, 0)


def _combine_body(a_ref, b_ref, o_ref):
    o_ref[...] = jax.nn.sigmoid(a_ref[...] + b_ref[...])


def _combine(ga, gb):
    r = 8000
    out = pl.pallas_call(
        _combine_body,
        grid=(E // r,),
        in_specs=[
            pl.BlockSpec((r, 16), lambda i: (i, 0)),
            pl.BlockSpec((r, 16), lambda i: (i, 0)),
        ],
        out_specs=pl.BlockSpec((r, 16), lambda i: (i, 0)),
        out_shape=jax.ShapeDtypeStruct((E, 16), jnp.float32),
    )(ga, gb)
    return out[:, 0]


def kernel(features, edge_index, W_in, b_in, msg_W, msg_b, gru_Wih, gru_Whh,
           gru_bih, gru_bhh, W_out, b_out):
    src = edge_index[0]
    dst = edge_index[1]
    src3 = src.reshape(NSUB, NEB, EB)
    dst3 = dst.reshape(NSUB, NEB, EB)
    zeros_rows = jnp.zeros((RPT, FC), jnp.float32)

    src_p = src.reshape(NW, ES_NB, ES_B)
    dst_p = dst.reshape(NW, ES_NB, ES_B)

    wih4 = [jnp.transpose(gru_Wih[l]).reshape(NCHUNK, FC, 3 * H)
            for l in range(L)]
    whh_t = [jnp.transpose(gru_Whh[l]) for l in range(L)]
    wab = jnp.zeros((H, 32), jnp.float32)
    wab = wab.at[:, 0].set(W_out[:H, 0]).at[:, 16].set(W_out[H:, 0])
    b32 = jnp.zeros((1, 32), jnp.float32).at[0, 0].set(b_out[0])

    x_p = jnp.zeros((NP, features.shape[1]), jnp.float32).at[:N].set(features)
    h, m4 = _in_proj(x_p, W_in, b_in, msg_W[0], msg_b[0])
    for l in range(L - 1):
        agg4 = _seg_sum(m4, src3, dst3, zeros_rows)
        gh = _gh(h, whh_t[l], gru_bhh[l])
        h, m4 = _gru(agg4, h, wih4[l], gh, gru_bih[l],
                     msg_W[l + 1], msg_b[l + 1])
    agg4 = _seg_sum(m4, src3, dst3, zeros_rows)
    gh = _gh(h, whh_t[L - 1], gru_bhh[L - 1])
    s = _gru_final(agg4, h, wih4[L - 1], gh, gru_bih[L - 1], wab, b32)

    a16 = s[:, :16]
    b16 = s[:, 16:]
    ga, gb = _edge_gather_kernel(a16, b16, src_p, dst_p)
    return _combine(ga, gb)


# speedup vs baseline: 1.1687x; 1.0027x over previous
import functools

import jax
import jax.numpy as jnp
from jax import lax
from jax.experimental import pallas as pl
from jax.experimental.pallas import tpu as pltpu
from jax.experimental.pallas import tpu_sc as plsc

N = 10000
E = 160000
H = 512
L = 3

NSC = 2
NSUB = 16
NW = NSC * NSUB

NP = 10240
RPT = NP // NSUB

FC = 128
NCHUNK = H // FC

EPT = E // NSUB
EB = 125
NEB = EPT // EB

ES_PER_W = E // NW
ES_B = 125
ES_NB = ES_PER_W // ES_B
ES_JB = 8
ES_NJ = ES_NB // ES_JB


def _chunked(h):
    return jnp.transpose(h.reshape(h.shape[0], NCHUNK, FC), (1, 0, 2))


def _in_proj_body(x_ref, w_ref, b_ref, mw_ref, mb_ref, o_ref, o4_ref):
    acc = jnp.dot(x_ref[...], w_ref[...], preferred_element_type=jnp.float32)
    h = jnp.maximum(acc + b_ref[...], 0.0)
    o_ref[...] = h
    m = jnp.dot(h, mw_ref[...], preferred_element_type=jnp.float32)
    o4_ref[...] = _chunked(m + mb_ref[...])


def _in_proj(x, w, b, mw, mb):
    d_in = x.shape[1]
    r = 2048
    return pl.pallas_call(
        _in_proj_body,
        grid=(NP // r,),
        in_specs=[
            pl.BlockSpec((r, d_in), lambda i: (i, 0)),
            pl.BlockSpec((d_in, H), lambda i: (0, 0)),
            pl.BlockSpec((1, H), lambda i: (0, 0)),
            pl.BlockSpec((H, H), lambda i: (0, 0)),
            pl.BlockSpec((1, H), lambda i: (0, 0)),
        ],
        out_specs=[
            pl.BlockSpec((r, H), lambda i: (i, 0)),
            pl.BlockSpec((NCHUNK, r, FC), lambda i: (0, i, 0)),
        ],
        out_shape=[
            jax.ShapeDtypeStruct((NP, H), jnp.float32),
            jax.ShapeDtypeStruct((NCHUNK, NP, FC), jnp.float32),
        ],
    )(x, w, b.reshape(1, H), mw, mb.reshape(1, H))


def _gh_body(h_ref, whh_ref, bhh_ref, o_ref):
    acc = jnp.dot(h_ref[...], whh_ref[...], preferred_element_type=jnp.float32)
    o_ref[...] = acc + bhh_ref[...]


def _gh(h, whh_t, bhh):
    r = 2048
    return pl.pallas_call(
        _gh_body,
        grid=(NP // r,),
        in_specs=[
            pl.BlockSpec((r, H), lambda i: (i, 0)),
            pl.BlockSpec((H, 3 * H), lambda i: (0, 0)),
            pl.BlockSpec((1, 3 * H), lambda i: (0, 0)),
        ],
        out_specs=pl.BlockSpec((r, 3 * H), lambda i: (i, 0)),
        out_shape=jax.ShapeDtypeStruct((NP, 3 * H), jnp.float32),
    )(h, whh_t, bhh.reshape(1, 3 * H))


def _gru_gates(agg_ref, h_ref, wih_ref, gh_ref, bih_ref):
    h = h_ref[...]
    gi = jnp.dot(agg_ref[0], wih_ref[0], preferred_element_type=jnp.float32)
    for c in range(1, NCHUNK):
        gi = gi + jnp.dot(agg_ref[c], wih_ref[c],
                          preferred_element_type=jnp.float32)
    gi = gi + bih_ref[...]
    gh = gh_ref[...]
    i_r, i_z, i_n = gi[:, :H], gi[:, H:2 * H], gi[:, 2 * H:]
    h_r, h_z, h_n = gh[:, :H], gh[:, H:2 * H], gh[:, 2 * H:]
    rg = jax.nn.sigmoid(i_r + h_r)
    z = jax.nn.sigmoid(i_z + h_z)
    n = jnp.tanh(i_n + rg * h_n)
    return (1.0 - z) * n + z * h


def _gru_body(agg_ref, h_ref, wih_ref, gh_ref, bih_ref,
              mw_ref, mb_ref, o_ref, o4_ref):
    hn = _gru_gates(agg_ref, h_ref, wih_ref, gh_ref, bih_ref)
    o_ref[...] = hn
    m = jnp.dot(hn, mw_ref[...], preferred_element_type=jnp.float32)
    o4_ref[...] = _chunked(m + mb_ref[...])


def _gru_final_body(agg_ref, h_ref, wih_ref, gh_ref, bih_ref,
                    wab_ref, b8_ref, s_ref):
    hn = _gru_gates(agg_ref, h_ref, wih_ref, gh_ref, bih_ref)
    s_ref[...] = jnp.dot(hn, wab_ref[...],
                         preferred_element_type=jnp.float32) + b8_ref[...]


def _gru(agg4, h, wih4, gh, bih, mw, mb):
    r = 1024
    return pl.pallas_call(
        _gru_body,
        grid=(NP // r,),
        in_specs=[
            pl.BlockSpec((NCHUNK, r, FC), lambda i: (0, i, 0)),
            pl.BlockSpec((r, H), lambda i: (i, 0)),
            pl.BlockSpec((NCHUNK, FC, 3 * H), lambda i: (0, 0, 0)),
            pl.BlockSpec((r, 3 * H), lambda i: (i, 0)),
            pl.BlockSpec((1, 3 * H), lambda i: (0, 0)),
            pl.BlockSpec((H, H), lambda i: (0, 0)),
            pl.BlockSpec((1, H), lambda i: (0, 0)),
        ],
        out_specs=[
            pl.BlockSpec((r, H), lambda i: (i, 0)),
            pl.BlockSpec((NCHUNK, r, FC), lambda i: (0, i, 0)),
        ],
        out_shape=[
            jax.ShapeDtypeStruct((NP, H), jnp.float32),
            jax.ShapeDtypeStruct((NCHUNK, NP, FC), jnp.float32),
        ],
    )(agg4, h, wih4, gh, bih.reshape(1, 3 * H), mw, mb.reshape(1, H))


def _gru_final(agg4, h, wih4, gh, bih, wab, b8):
    r = 1024
    return pl.pallas_call(
        _gru_final_body,
        grid=(NP // r,),
        in_specs=[
            pl.BlockSpec((NCHUNK, r, FC), lambda i: (0, i, 0)),
            pl.BlockSpec((r, H), lambda i: (i, 0)),
            pl.BlockSpec((NCHUNK, FC, 3 * H), lambda i: (0, 0, 0)),
            pl.BlockSpec((r, 3 * H), lambda i: (i, 0)),
            pl.BlockSpec((1, 3 * H), lambda i: (0, 0)),
            pl.BlockSpec((H, 32), lambda i: (0, 0)),
            pl.BlockSpec((1, 32), lambda i: (0, 0)),
        ],
        out_specs=pl.BlockSpec((r, 32), lambda i: (i, 0)),
        out_shape=jax.ShapeDtypeStruct((NP, 32), jnp.float32),
    )(agg4, h, wih4, gh, bih.reshape(1, 3 * H), wab, b8)


_MESH = plsc.VectorSubcoreMesh(core_axis_name="c", subcore_axis_name="s",
                               num_cores=NSC, num_subcores=NSUB)


NBUF = 2
HALF = NEB // 2
NGH = HALF // NBUF


@functools.partial(
    pl.kernel,
    out_type=jax.ShapeDtypeStruct((NCHUNK, NP, FC), jnp.float32),
    mesh=_MESH,
    scratch_types=[
        pltpu.VMEM((HALF, EB), jnp.int32),
        pltpu.VMEM((HALF, EB), jnp.int32),
        pltpu.VMEM((NBUF, EB, FC), jnp.float32),
        pltpu.VMEM_SHARED((NP, FC), jnp.float32),
        pltpu.SemaphoreType.DMA,
        pltpu.SemaphoreType.DMA,
        pltpu.SemaphoreType.DMA,
        pltpu.SemaphoreType.DMA,
    ],
)
def _seg_sum_kernel(m4_hbm, src_hbm, dst_hbm, zeros_hbm, out_hbm,
                    src_v, dst_v, rows_v, acc_sh, g0, g1, s0, s1):
    cid = lax.axis_index("c")
    sid = lax.axis_index("s")
    gsems = [g0, g1]
    ssems = [s0, s1]

    def process_chunk(chunk):
        pltpu.sync_copy(zeros_hbm, acc_sh.at[pl.ds(sid * RPT, RPT)])
        plsc.subcore_barrier()

        def start_gather(b, bi):
            pltpu.async_copy(m4_hbm.at[chunk].at[src_v.at[b]],
                             rows_v.at[bi], gsems[bi])

        def wait_gather(b, bi):
            pltpu.make_async_copy(m4_hbm.at[chunk].at[src_v.at[b]],
                                  rows_v.at[bi], gsems[bi]).wait()

        def start_scatter(b, bi):
            pltpu.async_copy(rows_v.at[bi], acc_sh.at[dst_v.at[b]],
                             ssems[bi], add=True)

        def wait_scatter(b, bi):
            pltpu.make_async_copy(rows_v.at[bi], acc_sh.at[dst_v.at[b]],
                                  ssems[bi]).wait()

        for half in range(2):
            pltpu.sync_copy(src_hbm.at[sid].at[pl.ds(half * HALF, HALF)],
                            src_v)
            pltpu.sync_copy(dst_hbm.at[sid].at[pl.ds(half * HALF, HALF)],
                            dst_v)

            for bi in range(NBUF):
                start_gather(bi, bi)

            def body(j, _):
                for bi in range(NBUF):
                    b = j * NBUF + bi
                    wait_gather(b, bi)
                    start_scatter(b, bi)
                    wait_scatter(b, bi)
                    start_gather(b + NBUF, bi)
                return 0

            lax.fori_loop(0, NGH - 1, body, 0)
            for bi in range(NBUF):
                b = (NGH - 1) * NBUF + bi
                wait_gather(b, bi)
                start_scatter(b, bi)
            for bi in range(NBUF):
                wait_scatter((NGH - 1) * NBUF + bi, bi)
        plsc.subcore_barrier()
        pltpu.sync_copy(acc_sh.at[pl.ds(sid * RPT, RPT)],
                        out_hbm.at[chunk].at[pl.ds(sid * RPT, RPT)])
        plsc.subcore_barrier()

    for cc in range(NCHUNK // NSC):
        for c0 in range(NSC):
            chunk = c0 * (NCHUNK // NSC) + cc

            @pl.when(cid == c0)
            def _():
                process_chunk(chunk)


def _seg_sum(m4, src3, dst3, zeros_rows):
    return _seg_sum_kernel(m4, src3, dst3, zeros_rows)


@functools.partial(
    pl.kernel,
    out_type=[
        jax.ShapeDtypeStruct((E, 8), jnp.float32),
        jax.ShapeDtypeStruct((E, 8), jnp.float32),
    ],
    mesh=_MESH,
    compiler_params=pltpu.CompilerParams(use_tc_tiling_on_sc=False),
    scratch_types=[
        pltpu.VMEM((ES_NB, ES_B), jnp.int32),
        pltpu.VMEM((ES_NB, ES_B), jnp.int32),
        pltpu.VMEM((ES_JB * ES_B, 8), jnp.float32),
        pltpu.VMEM((ES_JB * ES_B, 8), jnp.float32),
        pltpu.SemaphoreType.DMA,
        pltpu.SemaphoreType.DMA,
    ],
)
def _edge_gather_kernel(a_hbm, b_hbm, src_hbm, dst_hbm, ga_hbm, gb_hbm,
                        src_v, dst_v, stga_v, stgb_v, asem, bsem):
    cid = lax.axis_index("c")
    sid = lax.axis_index("s")
    wid = sid * NSC + cid
    pltpu.sync_copy(src_hbm.at[wid], src_v)
    pltpu.sync_copy(dst_hbm.at[wid], dst_v)

    def outer(j, _):
        def inner(i, _):
            b = j * ES_JB + i
            cpa = pltpu.async_copy(a_hbm.at[src_v.at[b]],
                                   stga_v.at[pl.ds(i * ES_B, ES_B)], asem)
            cpb = pltpu.async_copy(b_hbm.at[dst_v.at[b]],
                                   stgb_v.at[pl.ds(i * ES_B, ES_B)], bsem)
            cpa.wait()
            cpb.wait()
            return 0

        lax.fori_loop(0, ES_JB, inner, 0)
        base = wid * ES_PER_W + j * (ES_JB * ES_B)
        pltpu.sync_copy(stga_v, ga_hbm.at[pl.ds(base, ES_JB * ES_B)])
        pltpu.sync_copy(stgb_v, gb_hbm.at[pl.ds(base, ES_JB * ES_B)])
        return 0

    lax.fori_loop(0, ES_NJ, outer, 0)


def _combine_body(a_ref, b_ref, o_ref):
    o_ref[...] = jax.nn.sigmoid(a_ref[...] + b_ref[...])


def _combine(ga, gb):
    r = 8000
    out = pl.pallas_call(
        _combine_body,
        grid=(E // r,),
        in_specs=[
            pl.BlockSpec((r, 8), lambda i: (i, 0)),
            pl.BlockSpec((r, 8), lambda i: (i, 0)),
        ],
        out_specs=pl.BlockSpec((r, 8), lambda i: (i, 0)),
        out_shape=jax.ShapeDtypeStruct((E, 8), jnp.float32),
    )(ga, gb)
    return out[:, 0]


def kernel(features, edge_index, W_in, b_in, msg_W, msg_b, gru_Wih, gru_Whh,
           gru_bih, gru_bhh, W_out, b_out):
    src = edge_index[0]
    dst = edge_index[1]
    src3 = src.reshape(NSUB, NEB, EB)
    dst3 = dst.reshape(NSUB, NEB, EB)
    zeros_rows = jnp.zeros((RPT, FC), jnp.float32)

    src_p = src.reshape(NW, ES_NB, ES_B)
    dst_p = dst.reshape(NW, ES_NB, ES_B)

    wih4 = [jnp.transpose(gru_Wih[l]).reshape(NCHUNK, FC, 3 * H)
            for l in range(L)]
    whh_t = [jnp.transpose(gru_Whh[l]) for l in range(L)]
    wab = jnp.zeros((H, 32), jnp.float32)
    wab = wab.at[:, 0].set(W_out[:H, 0]).at[:, 16].set(W_out[H:, 0])
    b32 = jnp.zeros((1, 32), jnp.float32).at[0, 0].set(b_out[0])

    x_p = jnp.zeros((NP, features.shape[1]), jnp.float32).at[:N].set(features)
    h, m4 = _in_proj(x_p, W_in, b_in, msg_W[0], msg_b[0])
    for l in range(L - 1):
        agg4 = _seg_sum(m4, src3, dst3, zeros_rows)
        gh = _gh(h, whh_t[l], gru_bhh[l])
        h, m4 = _gru(agg4, h, wih4[l], gh, gru_bih[l],
                     msg_W[l + 1], msg_b[l + 1])
    agg4 = _seg_sum(m4, src3, dst3, zeros_rows)
    gh = _gh(h, whh_t[L - 1], gru_bhh[L - 1])
    s = _gru_final(agg4, h, wih4[L - 1], gh, gru_bih[L - 1], wab, b32)

    a8 = s[:, 0:8]
    b8 = s[:, 16:24]
    ga, gb = _edge_gather_kernel(a8, b8, src_p, dst_p)
    return _combine(ga, gb)


# speedup vs baseline: 1.1887x; 1.0171x over previous
import functools

import jax
import jax.numpy as jnp
from jax import lax
from jax.experimental import pallas as pl
from jax.experimental.pallas import tpu as pltpu
from jax.experimental.pallas import tpu_sc as plsc

N = 10000
E = 160000
H = 512
L = 3

NSC = 2
NSUB = 16
NW = NSC * NSUB

NP = 10240
RPT = NP // NSUB

FC = 128
NCHUNK = H // FC

EPT = E // NSUB
EB = 125
NEB = EPT // EB

ES_PER_W = E // NW
ES_B = 125
ES_NB = ES_PER_W // ES_B
ES_JB = 8
ES_NJ = ES_NB // ES_JB


def _chunked(h):
    return jnp.transpose(h.reshape(h.shape[0], NCHUNK, FC), (1, 0, 2))


def _in_proj_body(x_ref, w_ref, b_ref, mw_ref, mb_ref, o_ref, o4_ref):
    acc = jnp.dot(x_ref[...], w_ref[...], preferred_element_type=jnp.float32)
    h = jnp.maximum(acc + b_ref[...], 0.0)
    o_ref[...] = h
    m = jnp.dot(h, mw_ref[...], preferred_element_type=jnp.float32)
    o4_ref[...] = _chunked(m + mb_ref[...])


def _in_proj(x, w, b, mw, mb):
    d_in = x.shape[1]
    r = 2048
    return pl.pallas_call(
        _in_proj_body,
        grid=(NP // r,),
        in_specs=[
            pl.BlockSpec((r, d_in), lambda i: (i, 0)),
            pl.BlockSpec((d_in, H), lambda i: (0, 0)),
            pl.BlockSpec((1, H), lambda i: (0, 0)),
            pl.BlockSpec((H, H), lambda i: (0, 0)),
            pl.BlockSpec((1, H), lambda i: (0, 0)),
        ],
        out_specs=[
            pl.BlockSpec((r, H), lambda i: (i, 0)),
            pl.BlockSpec((NCHUNK, r, FC), lambda i: (0, i, 0)),
        ],
        out_shape=[
            jax.ShapeDtypeStruct((NP, H), jnp.float32),
            jax.ShapeDtypeStruct((NCHUNK, NP, FC), jnp.float32),
        ],
    )(x, w, b.reshape(1, H), mw, mb.reshape(1, H))


def _gh_body(h_ref, whh_ref, bhh_ref, o_ref):
    acc = jnp.dot(h_ref[...], whh_ref[...], preferred_element_type=jnp.float32)
    o_ref[...] = acc + bhh_ref[...]


def _gh(h, whh_t, bhh):
    r = 2048
    return pl.pallas_call(
        _gh_body,
        grid=(NP // r,),
        in_specs=[
            pl.BlockSpec((r, H), lambda i: (i, 0)),
            pl.BlockSpec((H, 3 * H), lambda i: (0, 0)),
            pl.BlockSpec((1, 3 * H), lambda i: (0, 0)),
        ],
        out_specs=pl.BlockSpec((r, 3 * H), lambda i: (i, 0)),
        out_shape=jax.ShapeDtypeStruct((NP, 3 * H), jnp.float32),
    )(h, whh_t, bhh.reshape(1, 3 * H))


def _gru_gates(agg_ref, h_ref, wih_ref, gh_ref, bih_ref):
    h = h_ref[...]
    gi = jnp.dot(agg_ref[0], wih_ref[0], preferred_element_type=jnp.float32)
    for c in range(1, NCHUNK):
        gi = gi + jnp.dot(agg_ref[c], wih_ref[c],
                          preferred_element_type=jnp.float32)
    gi = gi + bih_ref[...]
    gh = gh_ref[...]
    i_r, i_z, i_n = gi[:, :H], gi[:, H:2 * H], gi[:, 2 * H:]
    h_r, h_z, h_n = gh[:, :H], gh[:, H:2 * H], gh[:, 2 * H:]
    rg = jax.nn.sigmoid(i_r + h_r)
    z = jax.nn.sigmoid(i_z + h_z)
    n = jnp.tanh(i_n + rg * h_n)
    return (1.0 - z) * n + z * h


def _gru_body(agg_ref, h_ref, wih_ref, gh_ref, bih_ref,
              mw_ref, mb_ref, o_ref, o4_ref):
    hn = _gru_gates(agg_ref, h_ref, wih_ref, gh_ref, bih_ref)
    o_ref[...] = hn
    m = jnp.dot(hn, mw_ref[...], preferred_element_type=jnp.float32)
    o4_ref[...] = _chunked(m + mb_ref[...])


def _gru_final_body(agg_ref, h_ref, wih_ref, gh_ref, bih_ref,
                    wab_ref, b8_ref, s_ref):
    hn = _gru_gates(agg_ref, h_ref, wih_ref, gh_ref, bih_ref)
    s_ref[...] = jnp.dot(hn, wab_ref[...],
                         preferred_element_type=jnp.float32) + b8_ref[...]


def _gru(agg4, h, wih4, gh, bih, mw, mb):
    r = 1024
    return pl.pallas_call(
        _gru_body,
        grid=(NP // r,),
        in_specs=[
            pl.BlockSpec((NCHUNK, r, FC), lambda i: (0, i, 0)),
            pl.BlockSpec((r, H), lambda i: (i, 0)),
            pl.BlockSpec((NCHUNK, FC, 3 * H), lambda i: (0, 0, 0)),
            pl.BlockSpec((r, 3 * H), lambda i: (i, 0)),
            pl.BlockSpec((1, 3 * H), lambda i: (0, 0)),
            pl.BlockSpec((H, H), lambda i: (0, 0)),
            pl.BlockSpec((1, H), lambda i: (0, 0)),
        ],
        out_specs=[
            pl.BlockSpec((r, H), lambda i: (i, 0)),
            pl.BlockSpec((NCHUNK, r, FC), lambda i: (0, i, 0)),
        ],
        out_shape=[
            jax.ShapeDtypeStruct((NP, H), jnp.float32),
            jax.ShapeDtypeStruct((NCHUNK, NP, FC), jnp.float32),
        ],
    )(agg4, h, wih4, gh, bih.reshape(1, 3 * H), mw, mb.reshape(1, H))


def _gru_final(agg4, h, wih4, gh, bih, wab, b8):
    r = 1024
    return pl.pallas_call(
        _gru_final_body,
        grid=(NP // r,),
        in_specs=[
            pl.BlockSpec((NCHUNK, r, FC), lambda i: (0, i, 0)),
            pl.BlockSpec((r, H), lambda i: (i, 0)),
            pl.BlockSpec((NCHUNK, FC, 3 * H), lambda i: (0, 0, 0)),
            pl.BlockSpec((r, 3 * H), lambda i: (i, 0)),
            pl.BlockSpec((1, 3 * H), lambda i: (0, 0)),
            pl.BlockSpec((H, 32), lambda i: (0, 0)),
            pl.BlockSpec((1, 32), lambda i: (0, 0)),
        ],
        out_specs=pl.BlockSpec((r, 32), lambda i: (i, 0)),
        out_shape=jax.ShapeDtypeStruct((NP, 32), jnp.float32),
    )(agg4, h, wih4, gh, bih.reshape(1, 3 * H), wab, b8)


_MESH = plsc.VectorSubcoreMesh(core_axis_name="c", subcore_axis_name="s",
                               num_cores=NSC, num_subcores=NSUB)


NBUF = 2
HALF = NEB // 2
NGH = HALF // NBUF


@functools.partial(
    pl.kernel,
    out_type=jax.ShapeDtypeStruct((NCHUNK, NP, FC), jnp.float32),
    mesh=_MESH,
    scratch_types=[
        pltpu.VMEM((HALF, EB), jnp.int32),
        pltpu.VMEM((HALF, EB), jnp.int32),
        pltpu.VMEM((NBUF, EB, FC), jnp.float32),
        pltpu.VMEM_SHARED((NP, FC), jnp.float32),
        pltpu.SemaphoreType.DMA,
        pltpu.SemaphoreType.DMA,
        pltpu.SemaphoreType.DMA,
        pltpu.SemaphoreType.DMA,
    ],
)
def _seg_sum_kernel(m4_hbm, src_hbm, dst_hbm, zeros_hbm, out_hbm,
                    src_v, dst_v, rows_v, acc_sh, g0, g1, s0, s1):
    cid = lax.axis_index("c")
    sid = lax.axis_index("s")
    gsems = [g0, g1]
    ssems = [s0, s1]

    def process_chunk(chunk):
        pltpu.sync_copy(zeros_hbm, acc_sh.at[pl.ds(sid * RPT, RPT)])
        plsc.subcore_barrier()

        def start_gather(b, bi):
            pltpu.async_copy(m4_hbm.at[chunk].at[src_v.at[b]],
                             rows_v.at[bi], gsems[bi])

        def wait_gather(b, bi):
            pltpu.make_async_copy(m4_hbm.at[chunk].at[src_v.at[b]],
                                  rows_v.at[bi], gsems[bi]).wait()

        def start_scatter(b, bi):
            pltpu.async_copy(rows_v.at[bi], acc_sh.at[dst_v.at[b]],
                             ssems[bi], add=True)

        def wait_scatter(b, bi):
            pltpu.make_async_copy(rows_v.at[bi], acc_sh.at[dst_v.at[b]],
                                  ssems[bi]).wait()

        for half in range(2):
            pltpu.sync_copy(src_hbm.at[sid].at[pl.ds(half * HALF, HALF)],
                            src_v)
            pltpu.sync_copy(dst_hbm.at[sid].at[pl.ds(half * HALF, HALF)],
                            dst_v)

            for bi in range(NBUF):
                start_gather(bi, bi)

            def body(j, _):
                for bi in range(NBUF):
                    b = j * NBUF + bi
                    wait_gather(b, bi)
                    start_scatter(b, bi)
                    wait_scatter(b, bi)
                    start_gather(b + NBUF, bi)
                return 0

            lax.fori_loop(0, NGH - 1, body, 0)
            for bi in range(NBUF):
                b = (NGH - 1) * NBUF + bi
                wait_gather(b, bi)
                start_scatter(b, bi)
            for bi in range(NBUF):
                wait_scatter((NGH - 1) * NBUF + bi, bi)
        plsc.subcore_barrier()
        pltpu.sync_copy(acc_sh.at[pl.ds(sid * RPT, RPT)],
                        out_hbm.at[chunk].at[pl.ds(sid * RPT, RPT)])
        plsc.subcore_barrier()

    for cc in range(NCHUNK // NSC):
        for c0 in range(NSC):
            chunk = c0 * (NCHUNK // NSC) + cc

            @pl.when(cid == c0)
            def _():
                process_chunk(chunk)


def _seg_sum(m4, src3, dst3, zeros_rows):
    return _seg_sum_kernel(m4, src3, dst3, zeros_rows)


@functools.partial(
    pl.kernel,
    out_type=[
        jax.ShapeDtypeStruct((E, 8), jnp.float32),
        jax.ShapeDtypeStruct((E, 8), jnp.float32),
    ],
    mesh=_MESH,
    compiler_params=pltpu.CompilerParams(use_tc_tiling_on_sc=False),
    scratch_types=[
        pltpu.VMEM((ES_NB, ES_B), jnp.int32),
        pltpu.VMEM((ES_NB, ES_B), jnp.int32),
        pltpu.VMEM((ES_JB * ES_B, 8), jnp.float32),
        pltpu.VMEM((ES_JB * ES_B, 8), jnp.float32),
        pltpu.SemaphoreType.DMA,
        pltpu.SemaphoreType.DMA,
    ],
)
def _edge_gather_kernel(a_hbm, b_hbm, src_hbm, dst_hbm, ga_hbm, gb_hbm,
                        src_v, dst_v, stga_v, stgb_v, asem, bsem):
    cid = lax.axis_index("c")
    sid = lax.axis_index("s")
    wid = sid * NSC + cid
    pltpu.sync_copy(src_hbm.at[wid], src_v)
    pltpu.sync_copy(dst_hbm.at[wid], dst_v)

    def outer(j, _):
        def issue(i, _):
            b = j * ES_JB + i
            pltpu.async_copy(a_hbm.at[src_v.at[b]],
                             stga_v.at[pl.ds(i * ES_B, ES_B)], asem)
            pltpu.async_copy(b_hbm.at[dst_v.at[b]],
                             stgb_v.at[pl.ds(i * ES_B, ES_B)], bsem)
            return 0

        def drain(i, _):
            b = j * ES_JB + i
            pltpu.make_async_copy(a_hbm.at[src_v.at[b]],
                                  stga_v.at[pl.ds(i * ES_B, ES_B)],
                                  asem).wait()
            pltpu.make_async_copy(b_hbm.at[dst_v.at[b]],
                                  stgb_v.at[pl.ds(i * ES_B, ES_B)],
                                  bsem).wait()
            return 0

        lax.fori_loop(0, ES_JB, issue, 0)
        lax.fori_loop(0, ES_JB, drain, 0)
        base = wid * ES_PER_W + j * (ES_JB * ES_B)
        pltpu.sync_copy(stga_v, ga_hbm.at[pl.ds(base, ES_JB * ES_B)])
        pltpu.sync_copy(stgb_v, gb_hbm.at[pl.ds(base, ES_JB * ES_B)])
        return 0

    lax.fori_loop(0, ES_NJ, outer, 0)


def _combine_body(a_ref, b_ref, o_ref):
    o_ref[...] = jax.nn.sigmoid(a_ref[...] + b_ref[...])


def _combine(ga, gb):
    r = 8000
    out = pl.pallas_call(
        _combine_body,
        grid=(E // r,),
        in_specs=[
            pl.BlockSpec((r, 8), lambda i: (i, 0)),
            pl.BlockSpec((r, 8), lambda i: (i, 0)),
        ],
        out_specs=pl.BlockSpec((r, 8), lambda i: (i, 0)),
        out_shape=jax.ShapeDtypeStruct((E, 8), jnp.float32),
    )(ga, gb)
    return out[:, 0]


def kernel(features, edge_index, W_in, b_in, msg_W, msg_b, gru_Wih, gru_Whh,
           gru_bih, gru_bhh, W_out, b_out):
    src = edge_index[0]
    dst = edge_index[1]
    src3 = src.reshape(NSUB, NEB, EB)
    dst3 = dst.reshape(NSUB, NEB, EB)
    zeros_rows = jnp.zeros((RPT, FC), jnp.float32)

    src_p = src.reshape(NW, ES_NB, ES_B)
    dst_p = dst.reshape(NW, ES_NB, ES_B)

    wih4 = [jnp.transpose(gru_Wih[l]).reshape(NCHUNK, FC, 3 * H)
            for l in range(L)]
    whh_t = [jnp.transpose(gru_Whh[l]) for l in range(L)]
    wab = jnp.zeros((H, 32), jnp.float32)
    wab = wab.at[:, 0].set(W_out[:H, 0]).at[:, 16].set(W_out[H:, 0])
    b32 = jnp.zeros((1, 32), jnp.float32).at[0, 0].set(b_out[0])

    x_p = jnp.zeros((NP, features.shape[1]), jnp.float32).at[:N].set(features)
    h, m4 = _in_proj(x_p, W_in, b_in, msg_W[0], msg_b[0])
    for l in range(L - 1):
        agg4 = _seg_sum(m4, src3, dst3, zeros_rows)
        gh = _gh(h, whh_t[l], gru_bhh[l])
        h, m4 = _gru(agg4, h, wih4[l], gh, gru_bih[l],
                     msg_W[l + 1], msg_b[l + 1])
    agg4 = _seg_sum(m4, src3, dst3, zeros_rows)
    gh = _gh(h, whh_t[L - 1], gru_bhh[L - 1])
    s = _gru_final(agg4, h, wih4[L - 1], gh, gru_bih[L - 1], wab, b32)

    a8 = s[:, 0:8]
    b8 = s[:, 16:24]
    ga, gb = _edge_gather_kernel(a8, b8, src_p, dst_p)
    return _combine(ga, gb)
